# baseline (speedup 1.0000x reference)
import functools

import jax
import jax.numpy as jnp
import numpy as np
from jax import lax
from jax.experimental import pallas as pl
from jax.experimental.pallas import tpu as pltpu
from jax.experimental.pallas import tpu_sc as plsc

F = 26
D = 32
B = 1024
SR = F * D
F_PAD = 32
L = 16

NC, NS = 2, 16
NW = NC * NS
B_PER_W = B // NW

_II, _JJ = np.triu_indices(F, k=1)
PAIRS = [(int(i), int(j)) for i, j in zip(_II, _JJ)]


def _ffm_body(idx_hbm, wl_hbm, bias_hbm, table_hbm, out_hbm,
              wl_v, idx_v, rows_v, out_v, bias_v, sem):
    wid = lax.axis_index("s") * NC + lax.axis_index("c")
    base_b = wid * B_PER_W

    pltpu.sync_copy(wl_hbm, wl_v)
    pltpu.sync_copy(bias_hbm, bias_v)
    pltpu.sync_copy(idx_hbm.at[pl.ds(base_b, B_PER_W)], idx_v)
    bias_vec = bias_v[...]

    lane = lax.iota(jnp.int32, L)
    lin_maskf = (lane < (F - L)).astype(jnp.float32)

    def body(bb, lanevec):
        pltpu.async_copy(
            table_hbm.at[idx_v.at[bb]], rows_v, sem).wait()

        acc0 = jnp.zeros((L,), jnp.float32)
        acc1 = jnp.zeros((L,), jnp.float32)
        for i, j in PAIRS:
            acc0 = acc0 + rows_v[j, pl.ds(i * D, L)] * rows_v[i, pl.ds(j * D, L)]
            acc1 = acc1 + rows_v[j, pl.ds(i * D + L, L)] * rows_v[i, pl.ds(j * D + L, L)]

        ridx0 = idx_v[bb, pl.ds(0, L)]
        ridx1 = idx_v[bb, pl.ds(L, L)]
        lin0 = plsc.load_gather(wl_v, [ridx0])
        lin1 = plsc.load_gather(wl_v, [ridx1]) * lin_maskf

        total = jnp.sum(acc0 + acc1 + lin0 + lin1)

        lanevec = jnp.where(lane == (bb % L), total, lanevec)

        @pl.when(bb % L == L - 1)
        def _():
            s = lanevec + bias_vec
            sig = 1.0 / (1.0 + jnp.exp(-s))
            out_v[bb // L] = sig

        return lanevec

    lax.fori_loop(0, B_PER_W, body, jnp.zeros((L,), jnp.float32), unroll=False)

    pltpu.sync_copy(out_v, out_hbm.at[pl.ds(wid * (B_PER_W // L), B_PER_W // L)])


@jax.jit
def _ffm_sc(idx_pad, wl_flat, bias_bcast, table_packed):
    kfn = functools.partial(
        pl.kernel,
        out_type=jax.ShapeDtypeStruct((B // L, L), jnp.float32),
        mesh=plsc.VectorSubcoreMesh(core_axis_name="c", subcore_axis_name="s"),
        compiler_params=pltpu.CompilerParams(
            use_tc_tiling_on_sc=False, needs_layout_passes=False),
        scratch_types=[
            pltpu.VMEM((26000,), jnp.float32),
            pltpu.VMEM((B_PER_W, F_PAD), jnp.int32),
            pltpu.VMEM((F_PAD, SR), jnp.float32),
            pltpu.VMEM((B_PER_W // L, L), jnp.float32),
            pltpu.VMEM((L,), jnp.float32),
            pltpu.SemaphoreType.DMA,
        ],
    )(_ffm_body)
    return kfn(idx_pad, wl_flat, bias_bcast, table_packed)


def kernel(x, W_lin, bias, ffm_tables):
    f = x.shape[1]
    total = W_lin.shape[0]
    offsets = (jnp.arange(f, dtype=x.dtype) * (total // f))[None, :]
    idx = x + offsets
    idx = jnp.pad(idx, ((0, 0), (0, F_PAD - f)))
    packed = jnp.swapaxes(ffm_tables, 0, 1).reshape(total, f * ffm_tables.shape[-1])
    out = _ffm_sc(idx, W_lin.reshape(-1),
                  jnp.broadcast_to(bias, (L,)), packed)
    return out.reshape(-1)


# speedup vs baseline: 1.5342x; 1.5342x over previous
import functools

import jax
import jax.numpy as jnp
import numpy as np
from jax import lax
from jax.experimental import pallas as pl
from jax.experimental.pallas import tpu as pltpu
from jax.experimental.pallas import tpu_sc as plsc

F = 26
D = 32
B = 1024
ROWS = F * F
ROWS_PAD = 680
L = 16

NC, NS = 2, 16
NW = NC * NS
B_PER_W = B // NW

CHUNKS = [(c * 128, min(128, ROWS_PAD - c * 128)) for c in range((ROWS_PAD + 127) // 128)]

_II, _JJ = np.triu_indices(F, k=1)
PAIRS = [(int(i), int(j)) for i, j in zip(_II, _JJ)]


def _ffm_body(idx_hbm, wl_hbm, bias_hbm, table_hbm, out_hbm,
              wl_v, idx_v, rows_v, out_v, bias_v, sem0, sem1):
    wid = lax.axis_index("s") * NC + lax.axis_index("c")
    base_b = wid * B_PER_W
    sems = (sem0, sem1)

    pltpu.sync_copy(wl_hbm, wl_v)
    pltpu.sync_copy(bias_hbm, bias_v)
    pltpu.sync_copy(idx_hbm.at[pl.ds(base_b, B_PER_W)], idx_v)
    bias_vec = bias_v[...]

    lane = lax.iota(jnp.int32, L)
    lin_maskf = (lane < (F - L)).astype(jnp.float32)

    def issue(bb, slot):
        for off, n in CHUNKS:
            pltpu.async_copy(
                table_hbm.at[idx_v.at[bb, pl.ds(off, n)]],
                rows_v.at[slot, pl.ds(off, n)], sems[slot])

    def drain(bb, slot):
        for off, n in CHUNKS:
            pltpu.make_async_copy(
                table_hbm.at[idx_v.at[bb, pl.ds(off, n)]],
                rows_v.at[slot, pl.ds(off, n)], sems[slot]).wait()

    def compute(bb, slot, lanevec):
        acc0 = jnp.zeros((L,), jnp.float32)
        acc1 = jnp.zeros((L,), jnp.float32)
        for i, j in PAIRS:
            a = i * F + j
            p = j * F + i
            acc0 = acc0 + rows_v[slot, a, pl.ds(0, L)] * rows_v[slot, p, pl.ds(0, L)]
            acc1 = acc1 + rows_v[slot, a, pl.ds(L, L)] * rows_v[slot, p, pl.ds(L, L)]

        ridx0 = idx_v[bb, pl.ds(0, L)]
        ridx1 = jnp.where(lane < (F - L), idx_v[bb, pl.ds(L, L)], 0)
        lin0 = plsc.load_gather(wl_v, [ridx0])
        lin1 = plsc.load_gather(wl_v, [ridx1]) * lin_maskf

        total = jnp.sum(acc0 + acc1 + lin0 + lin1)

        lanevec = jnp.where(lane == (bb % L), total, lanevec)

        @pl.when(bb % L == L - 1)
        def _():
            s = lanevec + bias_vec
            sig = 1.0 / (1.0 + jnp.exp(-s))
            out_v[bb // L] = sig

        return lanevec

    issue(0, 0)

    def body2(k, lanevec):
        bb0 = 2 * k
        bb1 = bb0 + 1
        issue(bb1, 1)
        drain(bb0, 0)
        lanevec = compute(bb0, 0, lanevec)

        @pl.when(k < B_PER_W // 2 - 1)
        def _():
            issue(bb0 + 2, 0)

        drain(bb1, 1)
        lanevec = compute(bb1, 1, lanevec)
        return lanevec

    lax.fori_loop(0, B_PER_W // 2, body2, jnp.zeros((L,), jnp.float32),
                  unroll=False)

    pltpu.sync_copy(out_v, out_hbm.at[pl.ds(wid * (B_PER_W // L), B_PER_W // L)])


@jax.jit
def _ffm_sc(idx_pad, wl_flat, bias_bcast, table_flat):
    kfn = functools.partial(
        pl.kernel,
        out_type=jax.ShapeDtypeStruct((B // L, L), jnp.float32),
        mesh=plsc.VectorSubcoreMesh(core_axis_name="c", subcore_axis_name="s"),
        compiler_params=pltpu.CompilerParams(
            use_tc_tiling_on_sc=False, needs_layout_passes=False),
        scratch_types=[
            pltpu.VMEM((26000,), jnp.float32),
            pltpu.VMEM((B_PER_W, ROWS_PAD), jnp.int32),
            pltpu.VMEM((2, ROWS_PAD, D), jnp.float32),
            pltpu.VMEM((B_PER_W // L, L), jnp.float32),
            pltpu.VMEM((L,), jnp.float32),
            pltpu.SemaphoreType.DMA,
            pltpu.SemaphoreType.DMA,
        ],
    )(_ffm_body)
    return kfn(idx_pad, wl_flat, bias_bcast, table_flat)


def kernel(x, W_lin, bias, ffm_tables):
    f = x.shape[1]
    total = W_lin.shape[0]
    offsets = (jnp.arange(f, dtype=x.dtype) * (total // f))[None, :]
    idx = x + offsets
    g = (idx[:, None, :] + (jnp.arange(f, dtype=x.dtype) * total)[None, :, None])
    g = g.reshape(x.shape[0], f * f)
    g = jnp.pad(g, ((0, 0), (0, ROWS_PAD - f * f)))
    out = _ffm_sc(g, W_lin.reshape(-1),
                  jnp.broadcast_to(bias, (L,)),
                  ffm_tables.reshape(-1, ffm_tables.shape[-1]))
    return out.reshape(-1)


# speedup vs baseline: 1.5704x; 1.0236x over previous
import functools

import jax
import jax.numpy as jnp
import numpy as np
from jax import lax
from jax.experimental import pallas as pl
from jax.experimental.pallas import tpu as pltpu
from jax.experimental.pallas import tpu_sc as plsc

F = 26
D = 32
DH = 16
B = 1024
VB = 1000
L = 16

NC, NS = 2, 16
NW = NC * NS
NG = B // L

_II, _JJ = np.triu_indices(F, k=1)
N_UNITS = 2 * len(_II) + F
NU_PAD = ((N_UNITS + NW - 1) // NW) * NW
NU_BASE = N_UNITS // NW
NU_EXTRA = N_UNITS - NU_BASE * NW


def _build_meta() -> np.ndarray:
    m = np.zeros((NU_PAD, 16), np.int32)
    u = 0
    for i, j in zip(_II, _JJ):
        for dh in range(2):
            m[u, :8] = (j, i * VB, dh * DH, 0, i, j * VB, i, j)
            u += 1
    for f in range(F):
        m[u, :8] = (0, f * VB, 0, 1, 0, 0, f, 0)
        u += 1
    m[u:, 3] = 1
    return m


_META_NP = _build_meta()


def _ffm_body(meta_hbm, xt_hbm, wl_hbm, table_hbm, part_hbm,
              meta_v, rows_v, wband_v, xcol_v, acc_v, sem0, sem1):
    wid = lax.axis_index("s") * NC + lax.axis_index("c")
    sems = (sem0, sem1)

    pltpu.sync_copy(meta_hbm, meta_v)

    zero = jnp.zeros((L,), jnp.float32)

    def zbody(g, c):
        acc_v[g] = zero
        return c

    lax.fori_loop(0, NG, zbody, 0, unroll=False)

    def issue(u, slot):
        mrow = meta_v[u]
        kind = mrow[3]
        sem = sems[slot]

        @pl.when(kind == 0)
        def _():
            ta = mrow[0]
            va = pl.multiple_of(mrow[1], 8)
            d0 = pl.multiple_of(mrow[2], 16)
            tb = mrow[4]
            vb = pl.multiple_of(mrow[5], 8)
            ca, cb = mrow[6], mrow[7]
            pltpu.async_copy(table_hbm.at[ta, pl.ds(va, VB), pl.ds(d0, DH)],
                             rows_v.at[2 * slot], sem)
            pltpu.async_copy(table_hbm.at[tb, pl.ds(vb, VB), pl.ds(d0, DH)],
                             rows_v.at[2 * slot + 1], sem)
            pltpu.async_copy(xt_hbm.at[ca], xcol_v.at[slot, 0], sem)
            pltpu.async_copy(xt_hbm.at[cb], xcol_v.at[slot, 1], sem)

        @pl.when(kind == 1)
        def _():
            va, ca = pl.multiple_of(mrow[1], 8), mrow[6]
            pltpu.async_copy(wl_hbm.at[pl.ds(va, VB)], wband_v.at[slot], sem)
            pltpu.async_copy(xt_hbm.at[ca], xcol_v.at[slot, 0], sem)

    def drain(u, slot):
        mrow = meta_v[u]
        kind = mrow[3]
        sem = sems[slot]

        @pl.when(kind == 0)
        def _():
            ta = mrow[0]
            va = pl.multiple_of(mrow[1], 8)
            d0 = pl.multiple_of(mrow[2], 16)
            tb = mrow[4]
            vb = pl.multiple_of(mrow[5], 8)
            ca, cb = mrow[6], mrow[7]
            pltpu.make_async_copy(table_hbm.at[ta, pl.ds(va, VB), pl.ds(d0, DH)],
                                  rows_v.at[2 * slot], sem).wait()
            pltpu.make_async_copy(table_hbm.at[tb, pl.ds(vb, VB), pl.ds(d0, DH)],
                                  rows_v.at[2 * slot + 1], sem).wait()
            pltpu.make_async_copy(xt_hbm.at[ca], xcol_v.at[slot, 0], sem).wait()
            pltpu.make_async_copy(xt_hbm.at[cb], xcol_v.at[slot, 1], sem).wait()

        @pl.when(kind == 1)
        def _():
            va, ca = pl.multiple_of(mrow[1], 8), mrow[6]
            pltpu.make_async_copy(wl_hbm.at[pl.ds(va, VB)], wband_v.at[slot],
                                  sem).wait()
            pltpu.make_async_copy(xt_hbm.at[ca], xcol_v.at[slot, 0], sem).wait()

    dconsts = [jnp.full((L,), d, jnp.int32) for d in range(DH)]

    def compute(u, slot):
        kind = meta_v[u][3]

        @pl.when(kind == 0)
        def _():
            ra = rows_v.at[2 * slot]
            rb = rows_v.at[2 * slot + 1]

            def gbody(g, c):
                xi = xcol_v[slot, 0, pl.ds(g * L, L)]
                xj = xcol_v[slot, 1, pl.ds(g * L, L)]
                accg = acc_v[g]
                for d in range(DH):
                    av = plsc.load_gather(ra, [xi, dconsts[d]])
                    bv = plsc.load_gather(rb, [xj, dconsts[d]])
                    accg = accg + av * bv
                acc_v[g] = accg
                return c

            lax.fori_loop(0, NG, gbody, 0, unroll=False)

        @pl.when(kind == 1)
        def _():
            wb = wband_v.at[slot]

            def gbody(g, c):
                xf = xcol_v[slot, 0, pl.ds(g * L, L)]
                acc_v[g] = acc_v[g] + plsc.load_gather(wb, [xf])
                return c

            lax.fori_loop(0, NG, gbody, 0, unroll=False)

    nu = NU_BASE + (wid < NU_EXTRA).astype(jnp.int32)

    issue(wid, 0)

    def body(s, c):
        u = wid + NW * s
        unext = u + NW

        @pl.when(jnp.logical_and(s + 1 < nu, (s + 1) % 2 == 0))
        def _():
            issue(unext, 0)

        @pl.when(jnp.logical_and(s + 1 < nu, (s + 1) % 2 == 1))
        def _():
            issue(unext, 1)

        @pl.when(s % 2 == 0)
        def _():
            drain(u, 0)
            compute(u, 0)

        @pl.when(s % 2 == 1)
        def _():
            drain(u, 1)
            compute(u, 1)

        return c

    lax.fori_loop(0, nu, body, 0, unroll=False)

    pltpu.sync_copy(acc_v, part_hbm.at[wid])


def _combine_body(part_hbm, bias_hbm, out_hbm, pall_v, bias_v, ob_v):
    wid = lax.axis_index("s") * NC + lax.axis_index("c")
    pltpu.sync_copy(part_hbm, pall_v)
    pltpu.sync_copy(bias_hbm, bias_v)
    bias_vec = bias_v[...]
    for r in range(2):
        row = 2 * wid + r

        def sbody(k, acc):
            return acc + pall_v[k, row]

        s = lax.fori_loop(0, NW, sbody, bias_vec, unroll=False)
        ob_v[r] = 1.0 / (1.0 + jnp.exp(-s))
    pltpu.sync_copy(ob_v, out_hbm.at[pl.ds(2 * wid, 2)])


_SC_PARAMS = pltpu.CompilerParams(
    use_tc_tiling_on_sc=False, needs_layout_passes=False)
_MESH = dict(mesh=plsc.VectorSubcoreMesh(core_axis_name="c",
                                         subcore_axis_name="s"))


@jax.jit
def _ffm_sc(meta, xt, wl_flat, bias_bcast, table3):
    part = functools.partial(
        pl.kernel,
        out_type=jax.ShapeDtypeStruct((NW, NG, L), jnp.float32),
        compiler_params=_SC_PARAMS,
        scratch_types=[
            pltpu.VMEM((NU_PAD, 16), jnp.int32),
            pltpu.VMEM((4, VB, DH), jnp.float32),
            pltpu.VMEM((2, VB), jnp.float32),
            pltpu.VMEM((2, 2, B), jnp.int32),
            pltpu.VMEM((NG, L), jnp.float32),
            pltpu.SemaphoreType.DMA,
            pltpu.SemaphoreType.DMA,
        ],
        **_MESH,
    )(_ffm_body)(meta, xt, wl_flat, table3)

    out = functools.partial(
        pl.kernel,
        out_type=jax.ShapeDtypeStruct((NG, L), jnp.float32),
        compiler_params=_SC_PARAMS,
        scratch_types=[
            pltpu.VMEM((NW, NG, L), jnp.float32),
            pltpu.VMEM((L,), jnp.float32),
            pltpu.VMEM((2, L), jnp.float32),
        ],
        **_MESH,
    )(_combine_body)(part, bias_bcast)
    return out


def kernel(x, W_lin, bias, ffm_tables):
    meta = jnp.asarray(_META_NP)
    xt = x.T
    out = _ffm_sc(meta, xt, W_lin.reshape(-1),
                  jnp.broadcast_to(bias, (L,)), ffm_tables)
    return out.reshape(-1)
